# baseline (speedup 1.0000x reference)
import functools

import jax
import jax.numpy as jnp
from jax import lax
from jax.experimental import pallas as pl
from jax.experimental.pallas import tpu as pltpu
from jax.experimental.pallas import tpu_sc as plsc

B, Q, C = 32, 2048, 512
K = 100
KPAD = 112
NEG = float("-inf")
IBIG = 2**30


_QB = 1024


def _tc_qmax_kernel(x_ref, m_ref):
    x = x_ref[...]
    m_ref[...] = jnp.max(x, axis=2)[:, None, None, :]


def _stage1(pred_logits):
    nqb = Q // _QB
    out = pl.pallas_call(
        _tc_qmax_kernel,
        grid=(B, nqb),
        in_specs=[pl.BlockSpec((1, _QB, C), lambda b, q: (b, q, 0))],
        out_specs=pl.BlockSpec((1, 1, 1, _QB), lambda b, q: (b, q, 0, 0)),
        out_shape=jax.ShapeDtypeStruct((B, nqb, 1, _QB), jnp.float32),
    )(pred_logits)
    return out


_NCHUNK = Q // 16


def _iota16():
    return lax.broadcasted_iota(jnp.int32, (16,), 0)


def _get1_f(ref, pos):
    off = (pos >> 4) << 4
    c = ref[pl.ds(off, 16)]
    return jnp.sum(jnp.where(_iota16() == (pos & 15), c, 0.0))


def _get1_i(ref, pos):
    off = (pos >> 4) << 4
    c = ref[pl.ds(off, 16)]
    return jnp.sum(jnp.where(_iota16() == (pos & 15), c, 0))


def _set1(ref, pos, val):
    off = (pos >> 4) << 4
    c = ref[pl.ds(off, 16)]
    ref[pl.ds(off, 16)] = jnp.where(_iota16() == (pos & 15), val, c)


def _scan128(ref, base):
    accv = ref[pl.ds(base, 16)]
    acci = _iota16()
    for jj in range(1, 8):
        v = ref[pl.ds(base + jj * 16, 16)]
        upd = v > accv
        acci = jnp.where(upd, _iota16() + jj * 16, acci)
        accv = jnp.where(upd, v, accv)
    m = jnp.max(accv)
    pos = jnp.min(jnp.where(accv == m, acci, IBIG))
    return m, pos


def _scan_row512(rows_ref, slot):
    accv = rows_ref[slot, pl.ds(0, 16)]
    acci = _iota16()
    for jj in range(1, 32):
        v = rows_ref[slot, pl.ds(jj * 16, 16)]
        upd = v > accv
        acci = jnp.where(upd, _iota16() + jj * 16, acci)
        accv = jnp.where(upd, v, accv)
    m = jnp.max(accv)
    pos = jnp.min(jnp.where(accv == m, acci, IBIG))
    return m, pos


def _scan_row512_max(rows_ref, slot):
    accv = rows_ref[slot, pl.ds(0, 16)]
    for jj in range(1, 32):
        accv = jnp.maximum(accv, rows_ref[slot, pl.ds(jj * 16, 16)])
    return jnp.max(accv)


def _pick(cm_ref, ca_ref):
    cm = cm_ref[...]
    m = jnp.max(cm)
    qv = ca_ref[...] * 16 + _iota16()
    qwin = jnp.min(jnp.where(cm == m, qv, IBIG))
    return m, qwin


def _recompute_lane(qT_ref, cm_ref, ca_ref, l):
    m, pos = _scan128(qT_ref, l * 128)
    lanes = _iota16()
    cm_ref[...] = jnp.where(lanes == l, m, cm_ref[...])
    ca_ref[...] = jnp.where(lanes == l, pos, ca_ref[...])


def _build_transpose(src_ref, dst_ref):
    def body(i, c):
        v = src_ref[pl.ds(i * 16, 16)]
        plsc.store_scatter(dst_ref, [_iota16() * 128 + i], v)
        return c
    lax.fori_loop(0, _NCHUNK, body, jnp.int32(0))


def _build_colmax(src_ref, cm_ref, ca_ref):
    def body(i, carry):
        accv, acci = carry
        v = src_ref[pl.ds(i * 16, 16)]
        upd = v > accv
        return (jnp.where(upd, v, accv), jnp.where(upd, i, acci))
    accv0 = src_ref[pl.ds(0, 16)]
    acci0 = jnp.zeros((16,), jnp.int32)
    accv, acci = lax.fori_loop(1, _NCHUNK, body, (accv0, acci0))
    cm_ref[...] = accv
    ca_ref[...] = acci


def _sc_body(qmax_hbm, logits_hbm, boxes_hbm, tsz_hbm,
             sco_hbm, lab_hbm, qid_hbm, box_hbm,
             qmax, qmax2, qT, qT2, slot_map, boxes, rows,
             cm, ca, cm2, ca2, qsel, qselg, vout, qout, cout,
             tsz, sco, labv, qidv, boxo, sem):
    wid = lax.axis_index("s") * 2 + lax.axis_index("c")
    lanes = _iota16()

    pltpu.sync_copy(qmax_hbm.at[pl.ds(wid * Q, Q)], qmax)
    pltpu.sync_copy(boxes_hbm.at[pl.ds(wid * (2 * Q), 2 * Q)], boxes)
    pltpu.sync_copy(tsz_hbm, tsz)

    def cp_body(i, c):
        qmax2[pl.ds(i * 16, 16)] = qmax[pl.ds(i * 16, 16)]
        return c
    lax.fori_loop(0, _NCHUNK, cp_body, jnp.int32(0))

    _build_transpose(qmax, qT)
    _build_transpose(qmax2, qT2)
    _build_colmax(qmax, cm, ca)
    _build_colmax(qmax2, cm2, ca2)

    for c0 in range(KPAD // 16):
        qsel[pl.ds(c0 * 16, 16)] = jnp.zeros((16,), jnp.int32)
        qout[pl.ds(c0 * 16, 16)] = jnp.zeros((16,), jnp.int32)
        cout[pl.ds(c0 * 16, 16)] = jnp.zeros((16,), jnp.int32)
        vout[pl.ds(c0 * 16, 16)] = jnp.zeros((16,), jnp.float32)

    def s2_body(t, c):
        _, qwin = _pick(cm2, ca2)
        _set1(qsel, t, qwin)
        l = qwin & 15
        i = qwin >> 4
        _set1(qmax2, qwin, NEG)
        _set1(qT2, l * 128 + i, NEG)
        _recompute_lane(qT2, cm2, ca2, l)
        return c
    lax.fori_loop(0, K, s2_body, jnp.int32(0))

    base = wid * Q
    for c0 in range(KPAD // 16):
        qselg[pl.ds(c0 * 16, 16)] = qsel[pl.ds(c0 * 16, 16)] + base
        plsc.store_scatter(slot_map, [qsel[pl.ds(c0 * 16, 16)]],
                           lanes + c0 * 16)
    pltpu.async_copy(logits_hbm.at[qselg], rows, sem).wait()

    def s3_body(t, c):
        m, qwin = _pick(cm, ca)
        slot = _get1_i(slot_map, qwin)
        _, cls = _scan_row512(rows, slot)
        _set1(vout, t, m)
        _set1(qout, t, qwin)
        _set1(cout, t, cls)
        off = (cls >> 4) << 4
        ch = rows[slot, pl.ds(off, 16)]
        rows[slot, pl.ds(off, 16)] = jnp.where(lanes == (cls & 15), NEG, ch)
        m2 = _scan_row512_max(rows, slot)
        _set1(qmax, qwin, m2)
        l = qwin & 15
        i = qwin >> 4
        _set1(qT, l * 128 + i, m2)
        _recompute_lane(qT, cm, ca, l)
        return c
    lax.fori_loop(0, K, s3_body, jnp.int32(0))

    tchunk = tsz[pl.ds((wid >> 4) << 4, 16)]
    scale = jnp.sum(jnp.where(lanes == (wid & 15), tchunk, 0.0))
    for c0 in range(KPAD // 16):
        sl = pl.ds(c0 * 16, 16)
        v = vout[sl]
        q = qout[sl]
        cls = cout[sl]
        sco[sl] = 1.0 / (1.0 + jnp.exp(-v))
        labv[sl] = cls
        qidv[sl] = q
        cc = plsc.load_gather(boxes, [q * 2])
        ww = plsc.load_gather(boxes, [q * 2 + 1])
        t1 = jnp.clip(cc - 0.5 * ww, 0.0, 1.0) * scale
        t2 = jnp.clip(cc + 0.5 * ww, 0.0, 1.0) * scale
        plsc.store_scatter(boxo, [(lanes + c0 * 16) * 2], t1)
        plsc.store_scatter(boxo, [(lanes + c0 * 16) * 2 + 1], t2)

    pltpu.sync_copy(sco, sco_hbm.at[pl.ds(wid * KPAD, KPAD)])
    pltpu.sync_copy(labv, lab_hbm.at[pl.ds(wid * KPAD, KPAD)])
    pltpu.sync_copy(qidv, qid_hbm.at[pl.ds(wid * KPAD, KPAD)])
    pltpu.sync_copy(boxo, box_hbm.at[pl.ds(wid * 2 * KPAD, 2 * KPAD)])


def _stage2(qmax, logits2d, boxes_flat, target_sizes):
    mesh = plsc.VectorSubcoreMesh(core_axis_name="c", subcore_axis_name="s")
    f32, i32 = jnp.float32, jnp.int32
    run = pl.kernel(
        _sc_body,
        out_type=[
            jax.ShapeDtypeStruct((B * KPAD,), f32),
            jax.ShapeDtypeStruct((B * KPAD,), i32),
            jax.ShapeDtypeStruct((B * KPAD,), i32),
            jax.ShapeDtypeStruct((B * 2 * KPAD,), f32),
        ],
        mesh=mesh,
        compiler_params=pltpu.CompilerParams(needs_layout_passes=False),
        scratch_types=[
            pltpu.VMEM((Q,), f32),
            pltpu.VMEM((Q,), f32),
            pltpu.VMEM((Q,), f32),
            pltpu.VMEM((Q,), f32),
            pltpu.VMEM((Q,), i32),
            pltpu.VMEM((2 * Q,), f32),
            pltpu.VMEM((KPAD, C), f32),
            pltpu.VMEM((16,), f32),
            pltpu.VMEM((16,), i32),
            pltpu.VMEM((16,), f32),
            pltpu.VMEM((16,), i32),
            pltpu.VMEM((KPAD,), i32),
            pltpu.VMEM((KPAD,), i32),
            pltpu.VMEM((KPAD,), f32),
            pltpu.VMEM((KPAD,), i32),
            pltpu.VMEM((KPAD,), i32),
            pltpu.VMEM((32,), f32),
            pltpu.VMEM((KPAD,), f32),
            pltpu.VMEM((KPAD,), i32),
            pltpu.VMEM((KPAD,), i32),
            pltpu.VMEM((2 * KPAD,), f32),
            pltpu.SemaphoreType.DMA,
        ],
    )
    return run(qmax, logits2d, boxes_flat, target_sizes)


@jax.jit
def kernel(pred_logits, pred_boxes, target_sizes):
    qmax = _stage1(pred_logits)
    logits2d = pred_logits.reshape(B * Q, C)
    sco, lab, qid, box = _stage2(
        qmax.reshape(-1), logits2d,
        pred_boxes.reshape(-1), target_sizes)
    scores = sco.reshape(B, KPAD)[:, :K]
    labels = lab.reshape(B, KPAD)[:, :K]
    tb_idx = qid.reshape(B, KPAD)[:, :K]
    boxes = box.reshape(B, KPAD, 2)[:, :K, :]
    return scores, labels, boxes, tb_idx


# speedup vs baseline: 1.1719x; 1.1719x over previous
import functools

import jax
import jax.numpy as jnp
from jax import lax
from jax.experimental import pallas as pl
from jax.experimental.pallas import tpu as pltpu
from jax.experimental.pallas import tpu_sc as plsc

B, Q, C = 32, 2048, 512
K = 100
KPAD = 112
NEG = float("-inf")
IBIG = 2**30


_BB = 2


def _tc_qmax_kernel(x_ref, m_ref):
    x = x_ref[...]
    m_ref[...] = jnp.max(x, axis=2)[:, None, :]


def _stage1(pred_logits):
    return pl.pallas_call(
        _tc_qmax_kernel,
        grid=(B // _BB,),
        in_specs=[pl.BlockSpec((_BB, Q, C), lambda b: (b, 0, 0))],
        out_specs=pl.BlockSpec((_BB, 1, Q), lambda b: (b, 0, 0)),
        out_shape=jax.ShapeDtypeStruct((B, 1, Q), jnp.float32),
    )(pred_logits)


_NCHUNK = Q // 16


def _iota16():
    return lax.broadcasted_iota(jnp.int32, (16,), 0)


def _get1_f(ref, pos):
    off = (pos >> 4) << 4
    c = ref[pl.ds(off, 16)]
    return jnp.sum(jnp.where(_iota16() == (pos & 15), c, 0.0))


def _get1_i(ref, pos):
    off = (pos >> 4) << 4
    c = ref[pl.ds(off, 16)]
    return jnp.sum(jnp.where(_iota16() == (pos & 15), c, 0))


def _set1(ref, pos, val):
    off = (pos >> 4) << 4
    c = ref[pl.ds(off, 16)]
    ref[pl.ds(off, 16)] = jnp.where(_iota16() == (pos & 15), val, c)


def _scan128(ref, base):
    accv = ref[pl.ds(base, 16)]
    acci = _iota16()
    for jj in range(1, 8):
        v = ref[pl.ds(base + jj * 16, 16)]
        upd = v > accv
        acci = jnp.where(upd, _iota16() + jj * 16, acci)
        accv = jnp.where(upd, v, accv)
    m = jnp.max(accv)
    pos = jnp.min(jnp.where(accv == m, acci, IBIG))
    return m, pos


def _scan_row512(rows_ref, slot):
    accv = rows_ref[slot, pl.ds(0, 16)]
    acci = _iota16()
    for jj in range(1, 32):
        v = rows_ref[slot, pl.ds(jj * 16, 16)]
        upd = v > accv
        acci = jnp.where(upd, _iota16() + jj * 16, acci)
        accv = jnp.where(upd, v, accv)
    m = jnp.max(accv)
    pos = jnp.min(jnp.where(accv == m, acci, IBIG))
    return m, pos


def _scan_row512_max(rows_ref, slot):
    accv = rows_ref[slot, pl.ds(0, 16)]
    for jj in range(1, 32):
        accv = jnp.maximum(accv, rows_ref[slot, pl.ds(jj * 16, 16)])
    return jnp.max(accv)


def _pick(cm_ref, ca_ref):
    cm = cm_ref[...]
    m = jnp.max(cm)
    qv = ca_ref[...] * 16 + _iota16()
    qwin = jnp.min(jnp.where(cm == m, qv, IBIG))
    return m, qwin


def _recompute_lane(qT_ref, cm_ref, ca_ref, l):
    m, pos = _scan128(qT_ref, l * 128)
    lanes = _iota16()
    cm_ref[...] = jnp.where(lanes == l, m, cm_ref[...])
    ca_ref[...] = jnp.where(lanes == l, pos, ca_ref[...])


def _build_transpose(src_ref, dst_ref):
    def body(i, c):
        v = src_ref[pl.ds(i * 16, 16)]
        plsc.store_scatter(dst_ref, [_iota16() * 128 + i], v)
        return c
    lax.fori_loop(0, _NCHUNK, body, jnp.int32(0))


def _build_colmax(src_ref, cm_ref, ca_ref):
    def body(i, carry):
        accv, acci = carry
        v = src_ref[pl.ds(i * 16, 16)]
        upd = v > accv
        return (jnp.where(upd, v, accv), jnp.where(upd, i, acci))
    accv0 = src_ref[pl.ds(0, 16)]
    acci0 = jnp.zeros((16,), jnp.int32)
    accv, acci = lax.fori_loop(1, _NCHUNK, body, (accv0, acci0))
    cm_ref[...] = accv
    ca_ref[...] = acci


def _sc_body(qmax_hbm, logits_hbm, boxes_hbm, tsz_hbm,
             sco_hbm, lab_hbm, qid_hbm, box_hbm,
             qmax, qmax2, qT, qT2, slot_map, boxes, rows,
             cm, ca, cm2, ca2, qsel, qselg, vout, qout, cout,
             tsz, sco, labv, qidv, boxo, sem):
    wid = lax.axis_index("s") * 2 + lax.axis_index("c")
    lanes = _iota16()

    pltpu.sync_copy(qmax_hbm.at[pl.ds(wid * Q, Q)], qmax)
    pltpu.sync_copy(boxes_hbm.at[pl.ds(wid * (2 * Q), 2 * Q)], boxes)
    pltpu.sync_copy(tsz_hbm, tsz)

    def cp_body(i, c):
        qmax2[pl.ds(i * 16, 16)] = qmax[pl.ds(i * 16, 16)]
        return c
    lax.fori_loop(0, _NCHUNK, cp_body, jnp.int32(0))

    _build_transpose(qmax, qT)
    _build_transpose(qmax2, qT2)
    _build_colmax(qmax, cm, ca)
    _build_colmax(qmax2, cm2, ca2)

    for c0 in range(KPAD // 16):
        qsel[pl.ds(c0 * 16, 16)] = jnp.zeros((16,), jnp.int32)
        qout[pl.ds(c0 * 16, 16)] = jnp.zeros((16,), jnp.int32)
        cout[pl.ds(c0 * 16, 16)] = jnp.zeros((16,), jnp.int32)
        vout[pl.ds(c0 * 16, 16)] = jnp.zeros((16,), jnp.float32)

    def s2_body(t, c):
        _, qwin = _pick(cm2, ca2)
        _set1(qsel, t, qwin)
        l = qwin & 15
        i = qwin >> 4
        _set1(qmax2, qwin, NEG)
        _set1(qT2, l * 128 + i, NEG)
        _recompute_lane(qT2, cm2, ca2, l)
        return c
    lax.fori_loop(0, K, s2_body, jnp.int32(0))

    base = wid * Q
    for c0 in range(KPAD // 16):
        qselg[pl.ds(c0 * 16, 16)] = qsel[pl.ds(c0 * 16, 16)] + base
        plsc.store_scatter(slot_map, [qsel[pl.ds(c0 * 16, 16)]],
                           lanes + c0 * 16)
    pltpu.async_copy(logits_hbm.at[qselg], rows, sem).wait()

    def s3_body(t, c):
        m, qwin = _pick(cm, ca)
        slot = _get1_i(slot_map, qwin)
        _, cls = _scan_row512(rows, slot)
        _set1(vout, t, m)
        _set1(qout, t, qwin)
        _set1(cout, t, cls)
        off = (cls >> 4) << 4
        ch = rows[slot, pl.ds(off, 16)]
        rows[slot, pl.ds(off, 16)] = jnp.where(lanes == (cls & 15), NEG, ch)
        m2 = _scan_row512_max(rows, slot)
        _set1(qmax, qwin, m2)
        l = qwin & 15
        i = qwin >> 4
        _set1(qT, l * 128 + i, m2)
        _recompute_lane(qT, cm, ca, l)
        return c
    lax.fori_loop(0, K, s3_body, jnp.int32(0))

    tchunk = tsz[pl.ds((wid >> 4) << 4, 16)]
    scale = jnp.sum(jnp.where(lanes == (wid & 15), tchunk, 0.0))
    for c0 in range(KPAD // 16):
        sl = pl.ds(c0 * 16, 16)
        v = vout[sl]
        q = qout[sl]
        cls = cout[sl]
        sco[sl] = 1.0 / (1.0 + jnp.exp(-v))
        labv[sl] = cls
        qidv[sl] = q
        cc = plsc.load_gather(boxes, [q * 2])
        ww = plsc.load_gather(boxes, [q * 2 + 1])
        t1 = jnp.clip(cc - 0.5 * ww, 0.0, 1.0) * scale
        t2 = jnp.clip(cc + 0.5 * ww, 0.0, 1.0) * scale
        plsc.store_scatter(boxo, [(lanes + c0 * 16) * 2], t1)
        plsc.store_scatter(boxo, [(lanes + c0 * 16) * 2 + 1], t2)

    pltpu.sync_copy(sco, sco_hbm.at[pl.ds(wid * KPAD, KPAD)])
    pltpu.sync_copy(labv, lab_hbm.at[pl.ds(wid * KPAD, KPAD)])
    pltpu.sync_copy(qidv, qid_hbm.at[pl.ds(wid * KPAD, KPAD)])
    pltpu.sync_copy(boxo, box_hbm.at[pl.ds(wid * 2 * KPAD, 2 * KPAD)])


def _stage2(qmax, logits2d, boxes_flat, target_sizes):
    mesh = plsc.VectorSubcoreMesh(core_axis_name="c", subcore_axis_name="s")
    f32, i32 = jnp.float32, jnp.int32
    run = pl.kernel(
        _sc_body,
        out_type=[
            jax.ShapeDtypeStruct((B * KPAD,), f32),
            jax.ShapeDtypeStruct((B * KPAD,), i32),
            jax.ShapeDtypeStruct((B * KPAD,), i32),
            jax.ShapeDtypeStruct((B * 2 * KPAD,), f32),
        ],
        mesh=mesh,
        compiler_params=pltpu.CompilerParams(needs_layout_passes=False),
        scratch_types=[
            pltpu.VMEM((Q,), f32),
            pltpu.VMEM((Q,), f32),
            pltpu.VMEM((Q,), f32),
            pltpu.VMEM((Q,), f32),
            pltpu.VMEM((Q,), i32),
            pltpu.VMEM((2 * Q,), f32),
            pltpu.VMEM((KPAD, C), f32),
            pltpu.VMEM((16,), f32),
            pltpu.VMEM((16,), i32),
            pltpu.VMEM((16,), f32),
            pltpu.VMEM((16,), i32),
            pltpu.VMEM((KPAD,), i32),
            pltpu.VMEM((KPAD,), i32),
            pltpu.VMEM((KPAD,), f32),
            pltpu.VMEM((KPAD,), i32),
            pltpu.VMEM((KPAD,), i32),
            pltpu.VMEM((32,), f32),
            pltpu.VMEM((KPAD,), f32),
            pltpu.VMEM((KPAD,), i32),
            pltpu.VMEM((KPAD,), i32),
            pltpu.VMEM((2 * KPAD,), f32),
            pltpu.SemaphoreType.DMA,
        ],
    )
    return run(qmax, logits2d, boxes_flat, target_sizes)


@jax.jit
def kernel(pred_logits, pred_boxes, target_sizes):
    qmax = _stage1(pred_logits)
    logits2d = pred_logits.reshape(B * Q, C)
    sco, lab, qid, box = _stage2(
        qmax.reshape(-1), logits2d,
        pred_boxes.reshape(-1), target_sizes)
    scores = sco.reshape(B, KPAD)[:, :K]
    labels = lab.reshape(B, KPAD)[:, :K]
    tb_idx = qid.reshape(B, KPAD)[:, :K]
    boxes = box.reshape(B, KPAD, 2)[:, :K, :]
    return scores, labels, boxes, tb_idx


# speedup vs baseline: 1.1937x; 1.0187x over previous
import functools

import jax
import jax.numpy as jnp
from jax import lax
from jax.experimental import pallas as pl
from jax.experimental.pallas import tpu as pltpu
from jax.experimental.pallas import tpu_sc as plsc

B, Q, C = 32, 2048, 512
K = 100
KPAD = 112
NEG = float("-inf")
IBIG = 2**30


_BB = 4


def _tc_qmax_kernel(x_ref, m_ref):
    x = x_ref[...]
    m_ref[...] = jnp.max(x, axis=2)[:, None, :]


def _stage1(pred_logits):
    return pl.pallas_call(
        _tc_qmax_kernel,
        grid=(B // _BB,),
        in_specs=[pl.BlockSpec((_BB, Q, C), lambda b: (b, 0, 0))],
        out_specs=pl.BlockSpec((_BB, 1, Q), lambda b: (b, 0, 0)),
        out_shape=jax.ShapeDtypeStruct((B, 1, Q), jnp.float32),
    )(pred_logits)


_NCHUNK = Q // 16


def _iota16():
    return lax.broadcasted_iota(jnp.int32, (16,), 0)


def _get1_f(ref, pos):
    off = (pos >> 4) << 4
    c = ref[pl.ds(off, 16)]
    return jnp.sum(jnp.where(_iota16() == (pos & 15), c, 0.0))


def _get1_i(ref, pos):
    off = (pos >> 4) << 4
    c = ref[pl.ds(off, 16)]
    return jnp.sum(jnp.where(_iota16() == (pos & 15), c, 0))


def _set1(ref, pos, val):
    off = (pos >> 4) << 4
    c = ref[pl.ds(off, 16)]
    ref[pl.ds(off, 16)] = jnp.where(_iota16() == (pos & 15), val, c)


def _scan128(ref, base):
    accv = ref[pl.ds(base, 16)]
    acci = _iota16()
    for jj in range(1, 8):
        v = ref[pl.ds(base + jj * 16, 16)]
        upd = v > accv
        acci = jnp.where(upd, _iota16() + jj * 16, acci)
        accv = jnp.where(upd, v, accv)
    m = jnp.max(accv)
    pos = jnp.min(jnp.where(accv == m, acci, IBIG))
    return m, pos


def _scan_row512(rows_ref, slot):
    accv = rows_ref[slot, pl.ds(0, 16)]
    acci = _iota16()
    for jj in range(1, 32):
        v = rows_ref[slot, pl.ds(jj * 16, 16)]
        upd = v > accv
        acci = jnp.where(upd, _iota16() + jj * 16, acci)
        accv = jnp.where(upd, v, accv)
    m = jnp.max(accv)
    pos = jnp.min(jnp.where(accv == m, acci, IBIG))
    return m, pos


def _scan_row512_max(rows_ref, slot):
    accv = rows_ref[slot, pl.ds(0, 16)]
    for jj in range(1, 32):
        accv = jnp.maximum(accv, rows_ref[slot, pl.ds(jj * 16, 16)])
    return jnp.max(accv)


def _pick(cm_ref, ca_ref):
    cm = cm_ref[...]
    m = jnp.max(cm)
    qv = ca_ref[...] * 16 + _iota16()
    qwin = jnp.min(jnp.where(cm == m, qv, IBIG))
    return m, qwin


def _recompute_lane(qT_ref, cm_ref, ca_ref, l):
    m, pos = _scan128(qT_ref, l * 128)
    lanes = _iota16()
    cm_ref[...] = jnp.where(lanes == l, m, cm_ref[...])
    ca_ref[...] = jnp.where(lanes == l, pos, ca_ref[...])


def _build_transpose(src_ref, dst_ref):
    def body(i, c):
        v = src_ref[pl.ds(i * 16, 16)]
        plsc.store_scatter(dst_ref, [_iota16() * 128 + i], v)
        return c
    lax.fori_loop(0, _NCHUNK, body, jnp.int32(0))


def _build_colmax(src_ref, cm_ref, ca_ref):
    def body(i, carry):
        accv, acci = carry
        v = src_ref[pl.ds(i * 16, 16)]
        upd = v > accv
        return (jnp.where(upd, v, accv), jnp.where(upd, i, acci))
    accv0 = src_ref[pl.ds(0, 16)]
    acci0 = jnp.zeros((16,), jnp.int32)
    accv, acci = lax.fori_loop(1, _NCHUNK, body, (accv0, acci0))
    cm_ref[...] = accv
    ca_ref[...] = acci


def _sc_body(qmax_hbm, logits_hbm, boxes_hbm, tsz_hbm,
             sco_hbm, lab_hbm, qid_hbm, box_hbm,
             qmax, qmax2, qT, qT2, slot_map, boxes, rows,
             cm, ca, cm2, ca2, qsel, qselg, vout, qout, cout,
             tsz, sco, labv, qidv, boxo, sem):
    wid = lax.axis_index("s") * 2 + lax.axis_index("c")
    lanes = _iota16()

    pltpu.sync_copy(qmax_hbm.at[pl.ds(wid * Q, Q)], qmax)
    pltpu.sync_copy(boxes_hbm.at[pl.ds(wid * (2 * Q), 2 * Q)], boxes)
    pltpu.sync_copy(tsz_hbm, tsz)

    def cp_body(i, c):
        qmax2[pl.ds(i * 16, 16)] = qmax[pl.ds(i * 16, 16)]
        return c
    lax.fori_loop(0, _NCHUNK, cp_body, jnp.int32(0))

    _build_transpose(qmax, qT)
    _build_transpose(qmax2, qT2)
    _build_colmax(qmax, cm, ca)
    _build_colmax(qmax2, cm2, ca2)

    for c0 in range(KPAD // 16):
        qsel[pl.ds(c0 * 16, 16)] = jnp.zeros((16,), jnp.int32)
        qout[pl.ds(c0 * 16, 16)] = jnp.zeros((16,), jnp.int32)
        cout[pl.ds(c0 * 16, 16)] = jnp.zeros((16,), jnp.int32)
        vout[pl.ds(c0 * 16, 16)] = jnp.zeros((16,), jnp.float32)

    def s2_body(t, c):
        _, qwin = _pick(cm2, ca2)
        _set1(qsel, t, qwin)
        l = qwin & 15
        i = qwin >> 4
        _set1(qmax2, qwin, NEG)
        _set1(qT2, l * 128 + i, NEG)
        _recompute_lane(qT2, cm2, ca2, l)
        return c
    lax.fori_loop(0, K, s2_body, jnp.int32(0))

    base = wid * Q
    for c0 in range(KPAD // 16):
        qselg[pl.ds(c0 * 16, 16)] = qsel[pl.ds(c0 * 16, 16)] + base
        plsc.store_scatter(slot_map, [qsel[pl.ds(c0 * 16, 16)]],
                           lanes + c0 * 16)
    pltpu.async_copy(logits_hbm.at[qselg], rows, sem).wait()

    def s3_body(t, c):
        m, qwin = _pick(cm, ca)
        slot = _get1_i(slot_map, qwin)
        _, cls = _scan_row512(rows, slot)
        _set1(vout, t, m)
        _set1(qout, t, qwin)
        _set1(cout, t, cls)
        off = (cls >> 4) << 4
        ch = rows[slot, pl.ds(off, 16)]
        rows[slot, pl.ds(off, 16)] = jnp.where(lanes == (cls & 15), NEG, ch)
        m2 = _scan_row512_max(rows, slot)
        _set1(qmax, qwin, m2)
        l = qwin & 15
        i = qwin >> 4
        _set1(qT, l * 128 + i, m2)
        _recompute_lane(qT, cm, ca, l)
        return c
    lax.fori_loop(0, K, s3_body, jnp.int32(0))

    tchunk = tsz[pl.ds((wid >> 4) << 4, 16)]
    scale = jnp.sum(jnp.where(lanes == (wid & 15), tchunk, 0.0))
    for c0 in range(KPAD // 16):
        sl = pl.ds(c0 * 16, 16)
        v = vout[sl]
        q = qout[sl]
        cls = cout[sl]
        sco[sl] = 1.0 / (1.0 + jnp.exp(-v))
        labv[sl] = cls
        qidv[sl] = q
        cc = plsc.load_gather(boxes, [q * 2])
        ww = plsc.load_gather(boxes, [q * 2 + 1])
        t1 = jnp.clip(cc - 0.5 * ww, 0.0, 1.0) * scale
        t2 = jnp.clip(cc + 0.5 * ww, 0.0, 1.0) * scale
        plsc.store_scatter(boxo, [(lanes + c0 * 16) * 2], t1)
        plsc.store_scatter(boxo, [(lanes + c0 * 16) * 2 + 1], t2)

    pltpu.sync_copy(sco, sco_hbm.at[pl.ds(wid * KPAD, KPAD)])
    pltpu.sync_copy(labv, lab_hbm.at[pl.ds(wid * KPAD, KPAD)])
    pltpu.sync_copy(qidv, qid_hbm.at[pl.ds(wid * KPAD, KPAD)])
    pltpu.sync_copy(boxo, box_hbm.at[pl.ds(wid * 2 * KPAD, 2 * KPAD)])


def _stage2(qmax, logits2d, boxes_flat, target_sizes):
    mesh = plsc.VectorSubcoreMesh(core_axis_name="c", subcore_axis_name="s")
    f32, i32 = jnp.float32, jnp.int32
    run = pl.kernel(
        _sc_body,
        out_type=[
            jax.ShapeDtypeStruct((B * KPAD,), f32),
            jax.ShapeDtypeStruct((B * KPAD,), i32),
            jax.ShapeDtypeStruct((B * KPAD,), i32),
            jax.ShapeDtypeStruct((B * 2 * KPAD,), f32),
        ],
        mesh=mesh,
        compiler_params=pltpu.CompilerParams(needs_layout_passes=False),
        scratch_types=[
            pltpu.VMEM((Q,), f32),
            pltpu.VMEM((Q,), f32),
            pltpu.VMEM((Q,), f32),
            pltpu.VMEM((Q,), f32),
            pltpu.VMEM((Q,), i32),
            pltpu.VMEM((2 * Q,), f32),
            pltpu.VMEM((KPAD, C), f32),
            pltpu.VMEM((16,), f32),
            pltpu.VMEM((16,), i32),
            pltpu.VMEM((16,), f32),
            pltpu.VMEM((16,), i32),
            pltpu.VMEM((KPAD,), i32),
            pltpu.VMEM((KPAD,), i32),
            pltpu.VMEM((KPAD,), f32),
            pltpu.VMEM((KPAD,), i32),
            pltpu.VMEM((KPAD,), i32),
            pltpu.VMEM((32,), f32),
            pltpu.VMEM((KPAD,), f32),
            pltpu.VMEM((KPAD,), i32),
            pltpu.VMEM((KPAD,), i32),
            pltpu.VMEM((2 * KPAD,), f32),
            pltpu.SemaphoreType.DMA,
        ],
    )
    return run(qmax, logits2d, boxes_flat, target_sizes)


@jax.jit
def kernel(pred_logits, pred_boxes, target_sizes):
    qmax = _stage1(pred_logits)
    logits2d = pred_logits.reshape(B * Q, C)
    sco, lab, qid, box = _stage2(
        qmax.reshape(-1), logits2d,
        pred_boxes.reshape(-1), target_sizes)
    scores = sco.reshape(B, KPAD)[:, :K]
    labels = lab.reshape(B, KPAD)[:, :K]
    tb_idx = qid.reshape(B, KPAD)[:, :K]
    boxes = box.reshape(B, KPAD, 2)[:, :K, :]
    return scores, labels, boxes, tb_idx


# speedup vs baseline: 1.6409x; 1.3746x over previous
import functools

import jax
import jax.numpy as jnp
from jax import lax
from jax.experimental import pallas as pl
from jax.experimental.pallas import tpu as pltpu
from jax.experimental.pallas import tpu_sc as plsc

B, Q, C = 32, 2048, 512
K = 100
KPAD = 112
NEG = float("-inf")
IBIG = 2**30


_BB = 4


def _tc_qmax_kernel(x_ref, m_ref):
    x = x_ref[...]
    m_ref[...] = jnp.max(x, axis=2)[:, None, :]


def _stage1(pred_logits):
    return pl.pallas_call(
        _tc_qmax_kernel,
        grid=(B // _BB,),
        in_specs=[pl.BlockSpec((_BB, Q, C), lambda b: (b, 0, 0))],
        out_specs=pl.BlockSpec((_BB, 1, Q), lambda b: (b, 0, 0)),
        out_shape=jax.ShapeDtypeStruct((B, 1, Q), jnp.float32),
    )(pred_logits)


_NCHUNK = Q // 16


def _iota16():
    return lax.broadcasted_iota(jnp.int32, (16,), 0)


def _get1_f(ref, pos):
    off = (pos >> 4) << 4
    c = ref[pl.ds(off, 16)]
    return jnp.sum(jnp.where(_iota16() == (pos & 15), c, 0.0))


def _get1_i(ref, pos):
    off = (pos >> 4) << 4
    c = ref[pl.ds(off, 16)]
    return jnp.sum(jnp.where(_iota16() == (pos & 15), c, 0))


def _set1(ref, pos, val):
    off = (pos >> 4) << 4
    c = ref[pl.ds(off, 16)]
    ref[pl.ds(off, 16)] = jnp.where(_iota16() == (pos & 15), val, c)


def _scan128(ref, base):
    accv = ref[pl.ds(base, 16)]
    acci = _iota16()
    for jj in range(1, 8):
        v = ref[pl.ds(base + jj * 16, 16)]
        upd = v > accv
        acci = jnp.where(upd, _iota16() + jj * 16, acci)
        accv = jnp.where(upd, v, accv)
    m = jnp.max(accv)
    pos = jnp.min(jnp.where(accv == m, acci, IBIG))
    return m, pos


def _scan_row512(rows_ref, slot):
    accv = rows_ref[slot, pl.ds(0, 16)]
    acci = _iota16()
    for jj in range(1, 32):
        v = rows_ref[slot, pl.ds(jj * 16, 16)]
        upd = v > accv
        acci = jnp.where(upd, _iota16() + jj * 16, acci)
        accv = jnp.where(upd, v, accv)
    m = jnp.max(accv)
    pos = jnp.min(jnp.where(accv == m, acci, IBIG))
    return m, pos


def _scan_row512_max(rows_ref, slot):
    accv = rows_ref[slot, pl.ds(0, 16)]
    for jj in range(1, 32):
        accv = jnp.maximum(accv, rows_ref[slot, pl.ds(jj * 16, 16)])
    return jnp.max(accv)


def _pick(cm_ref, ca_ref):
    cm = cm_ref[...]
    m = jnp.max(cm)
    qv = ca_ref[...] * 16 + _iota16()
    qwin = jnp.min(jnp.where(cm == m, qv, IBIG))
    return m, qwin


def _recompute_lane(qT_ref, cm_ref, ca_ref, l):
    m, pos = _scan128(qT_ref, l * 128)
    lanes = _iota16()
    cm_ref[...] = jnp.where(lanes == l, m, cm_ref[...])
    ca_ref[...] = jnp.where(lanes == l, pos, ca_ref[...])


def _build_transpose(src_ref, dst_ref):
    def body(i, c):
        v = src_ref[pl.ds(i * 16, 16)]
        plsc.store_scatter(dst_ref, [_iota16() * 128 + i], v)
        return c
    lax.fori_loop(0, _NCHUNK, body, jnp.int32(0))


def _build_colmax(src_ref, cm_ref, ca_ref):
    def body(i, carry):
        accv, acci = carry
        v = src_ref[pl.ds(i * 16, 16)]
        upd = v > accv
        return (jnp.where(upd, v, accv), jnp.where(upd, i, acci))
    accv0 = src_ref[pl.ds(0, 16)]
    acci0 = jnp.zeros((16,), jnp.int32)
    accv, acci = lax.fori_loop(1, _NCHUNK, body, (accv0, acci0))
    cm_ref[...] = accv
    ca_ref[...] = acci


def _sc_body(qmax_hbm, logits_hbm, boxes_hbm, tsz_hbm,
             sco_hbm, lab_hbm, qid_hbm, box_hbm,
             qmax, qmax2, qT, qT2, slot_map, boxes, rows,
             cm, ca, cm2, ca2, qsel, qselg, vout, qout, cout,
             tsz, sco, labv, qidv, boxo, sem):
    wid = lax.axis_index("s") * 2 + lax.axis_index("c")
    lanes = _iota16()

    pltpu.sync_copy(qmax_hbm.at[pl.ds(wid * Q, Q)], qmax)
    pltpu.sync_copy(boxes_hbm.at[pl.ds(wid * (2 * Q), 2 * Q)], boxes)
    pltpu.sync_copy(tsz_hbm, tsz)

    def cp_body(i, c):
        qmax2[pl.ds(i * 16, 16)] = qmax[pl.ds(i * 16, 16)]
        return c
    lax.fori_loop(0, _NCHUNK, cp_body, jnp.int32(0))

    _build_transpose(qmax, qT)
    _build_transpose(qmax2, qT2)
    _build_colmax(qmax, cm, ca)
    _build_colmax(qmax2, cm2, ca2)

    for c0 in range(KPAD // 16):
        qsel[pl.ds(c0 * 16, 16)] = jnp.zeros((16,), jnp.int32)
        qout[pl.ds(c0 * 16, 16)] = jnp.zeros((16,), jnp.int32)
        cout[pl.ds(c0 * 16, 16)] = jnp.zeros((16,), jnp.int32)
        vout[pl.ds(c0 * 16, 16)] = jnp.zeros((16,), jnp.float32)

    def s2_body(t, c):
        _, qwin = _pick(cm2, ca2)
        _set1(qsel, t, qwin)
        l = qwin & 15
        i = qwin >> 4
        _set1(qmax2, qwin, NEG)
        _set1(qT2, l * 128 + i, NEG)
        _recompute_lane(qT2, cm2, ca2, l)
        return c
    lax.fori_loop(0, K, s2_body, jnp.int32(0))

    base = wid * Q
    for c0 in range(KPAD // 16):
        qselg[pl.ds(c0 * 16, 16)] = qsel[pl.ds(c0 * 16, 16)] + base
        plsc.store_scatter(slot_map, [qsel[pl.ds(c0 * 16, 16)]],
                           lanes + c0 * 16)
    pltpu.async_copy(logits_hbm.at[qselg], rows, sem).wait()

    def s3_body(t, c):
        m, qwin = _pick(cm, ca)
        slot = _get1_i(slot_map, qwin)
        _, cls = _scan_row512(rows, slot)
        _set1(vout, t, m)
        _set1(qout, t, qwin)
        _set1(cout, t, cls)
        off = (cls >> 4) << 4
        ch = rows[slot, pl.ds(off, 16)]
        rows[slot, pl.ds(off, 16)] = jnp.where(lanes == (cls & 15), NEG, ch)
        m2 = _scan_row512_max(rows, slot)
        _set1(qmax, qwin, m2)
        l = qwin & 15
        i = qwin >> 4
        _set1(qT, l * 128 + i, m2)
        _recompute_lane(qT, cm, ca, l)
        return c
    lax.fori_loop(0, K, s3_body, jnp.int32(0))

    tchunk = tsz[pl.ds((wid >> 4) << 4, 16)]
    scale = jnp.sum(jnp.where(lanes == (wid & 15), tchunk, 0.0))
    for c0 in range(KPAD // 16):
        sl = pl.ds(c0 * 16, 16)
        v = vout[sl]
        q = qout[sl]
        cls = cout[sl]
        sco[sl] = 1.0 / (1.0 + jnp.exp(-v))
        labv[sl] = cls
        qidv[sl] = q
        cc = plsc.load_gather(boxes, [q])
        ww = plsc.load_gather(boxes, [q + Q])
        t1 = jnp.clip(cc - 0.5 * ww, 0.0, 1.0) * scale
        t2 = jnp.clip(cc + 0.5 * ww, 0.0, 1.0) * scale
        plsc.store_scatter(boxo, [(lanes + c0 * 16) * 2], t1)
        plsc.store_scatter(boxo, [(lanes + c0 * 16) * 2 + 1], t2)

    pltpu.sync_copy(sco, sco_hbm.at[pl.ds(wid * KPAD, KPAD)])
    pltpu.sync_copy(labv, lab_hbm.at[pl.ds(wid * KPAD, KPAD)])
    pltpu.sync_copy(qidv, qid_hbm.at[pl.ds(wid * KPAD, KPAD)])
    pltpu.sync_copy(boxo, box_hbm.at[pl.ds(wid * 2 * KPAD, 2 * KPAD)])


def _stage2(qmax, logits2d, boxes_flat, target_sizes):
    mesh = plsc.VectorSubcoreMesh(core_axis_name="c", subcore_axis_name="s")
    f32, i32 = jnp.float32, jnp.int32
    run = pl.kernel(
        _sc_body,
        out_type=[
            jax.ShapeDtypeStruct((B * KPAD,), f32),
            jax.ShapeDtypeStruct((B * KPAD,), i32),
            jax.ShapeDtypeStruct((B * KPAD,), i32),
            jax.ShapeDtypeStruct((B * 2 * KPAD,), f32),
        ],
        mesh=mesh,
        compiler_params=pltpu.CompilerParams(needs_layout_passes=False),
        scratch_types=[
            pltpu.VMEM((Q,), f32),
            pltpu.VMEM((Q,), f32),
            pltpu.VMEM((Q,), f32),
            pltpu.VMEM((Q,), f32),
            pltpu.VMEM((Q,), i32),
            pltpu.VMEM((2 * Q,), f32),
            pltpu.VMEM((KPAD, C), f32),
            pltpu.VMEM((16,), f32),
            pltpu.VMEM((16,), i32),
            pltpu.VMEM((16,), f32),
            pltpu.VMEM((16,), i32),
            pltpu.VMEM((KPAD,), i32),
            pltpu.VMEM((KPAD,), i32),
            pltpu.VMEM((KPAD,), f32),
            pltpu.VMEM((KPAD,), i32),
            pltpu.VMEM((KPAD,), i32),
            pltpu.VMEM((32,), f32),
            pltpu.VMEM((KPAD,), f32),
            pltpu.VMEM((KPAD,), i32),
            pltpu.VMEM((KPAD,), i32),
            pltpu.VMEM((2 * KPAD,), f32),
            pltpu.SemaphoreType.DMA,
        ],
    )
    return run(qmax, logits2d, boxes_flat, target_sizes)


@jax.jit
def kernel(pred_logits, pred_boxes, target_sizes):
    qmax = _stage1(pred_logits)
    logits2d = pred_logits.reshape(B * Q, C)
    boxes_cw = jnp.transpose(pred_boxes, (0, 2, 1))
    sco, lab, qid, box = _stage2(
        qmax.reshape(-1), logits2d,
        boxes_cw.reshape(-1), target_sizes)
    scores = sco.reshape(B, KPAD)[:, :K]
    labels = lab.reshape(B, KPAD)[:, :K]
    tb_idx = qid.reshape(B, KPAD)[:, :K]
    boxes = box.reshape(B, KPAD, 2)[:, :K, :]
    return scores, labels, boxes, tb_idx


# speedup vs baseline: 1.7237x; 1.0505x over previous
import functools

import jax
import jax.numpy as jnp
from jax import lax
from jax.experimental import pallas as pl
from jax.experimental.pallas import tpu as pltpu
from jax.experimental.pallas import tpu_sc as plsc

B, Q, C = 32, 2048, 512
K = 100
KPAD = 112
NEG = float("-inf")
IBIG = 2**30


_BB = 4


def _tc_qmax_kernel(x_ref, m_ref):
    x = x_ref[...]
    m_ref[...] = jnp.max(x, axis=2)[:, None, :]


def _stage1(pred_logits):
    return pl.pallas_call(
        _tc_qmax_kernel,
        grid=(B // _BB,),
        in_specs=[pl.BlockSpec((_BB, Q, C), lambda b: (b, 0, 0))],
        out_specs=pl.BlockSpec((_BB, 1, Q), lambda b: (b, 0, 0)),
        out_shape=jax.ShapeDtypeStruct((B, 1, Q), jnp.float32),
    )(pred_logits)


_NCHUNK = Q // 16


def _iota16():
    return lax.broadcasted_iota(jnp.int32, (16,), 0)


def _set1(ref, pos, val):
    off = (pos >> 4) << 4
    c = ref[pl.ds(off, 16)]
    ref[pl.ds(off, 16)] = jnp.where(_iota16() == (pos & 15), val, c)


def _scan128(ref, base):
    accv = ref[pl.ds(base, 16)]
    acci = _iota16()
    for jj in range(1, 8):
        v = ref[pl.ds(base + jj * 16, 16)]
        upd = v > accv
        acci = jnp.where(upd, _iota16() + jj * 16, acci)
        accv = jnp.where(upd, v, accv)
    m = jnp.max(accv)
    pos = jnp.min(jnp.where(accv == m, acci, IBIG))
    return m, pos


def _pick(cm_ref, ca_ref):
    cm = cm_ref[...]
    m = jnp.max(cm)
    qv = ca_ref[...] * 16 + _iota16()
    qwin = jnp.min(jnp.where(cm == m, qv, IBIG))
    return m, qwin


def _recompute_lane(qT_ref, cm_ref, ca_ref, l):
    m, pos = _scan128(qT_ref, l * 128)
    lanes = _iota16()
    cm_ref[...] = jnp.where(lanes == l, m, cm_ref[...])
    ca_ref[...] = jnp.where(lanes == l, pos, ca_ref[...])


def _build_transpose(src_ref, dst_ref):
    def body(i, c):
        v = src_ref[pl.ds(i * 16, 16)]
        plsc.store_scatter(dst_ref, [_iota16() * 128 + i], v)
        return c
    lax.fori_loop(0, _NCHUNK, body, jnp.int32(0))


def _build_colmax(src_ref, cm_ref, ca_ref):
    def body(i, carry):
        accv, acci = carry
        v = src_ref[pl.ds(i * 16, 16)]
        upd = v > accv
        return (jnp.where(upd, v, accv), jnp.where(upd, i, acci))
    accv0 = src_ref[pl.ds(0, 16)]
    acci0 = jnp.zeros((16,), jnp.int32)
    accv, acci = lax.fori_loop(1, _NCHUNK, body, (accv0, acci0))
    cm_ref[...] = accv
    ca_ref[...] = acci


def _sc_body(qmax_hbm, logits_hbm, boxes_hbm, tsz_hbm,
             sco_hbm, lab_hbm, qid_hbm, box_hbm,
             qmax, qT, boxes, rows,
             cm, ca, qsel, qselg, kvec, candval, colv, colp,
             vout, sout, cout,
             tsz, sco, labv, qidv, boxo, sem):
    wid = lax.axis_index("s") * 2 + lax.axis_index("c")
    lanes = _iota16()

    pltpu.sync_copy(qmax_hbm.at[pl.ds(wid * Q, Q)], qmax)
    pltpu.sync_copy(boxes_hbm.at[pl.ds(wid * (2 * Q), 2 * Q)], boxes)
    pltpu.sync_copy(tsz_hbm, tsz)

    _build_transpose(qmax, qT)
    _build_colmax(qmax, cm, ca)

    for c0 in range(KPAD // 16):
        sl = pl.ds(c0 * 16, 16)
        qsel[sl] = jnp.zeros((16,), jnp.int32)
        sout[sl] = jnp.zeros((16,), jnp.int32)
        cout[sl] = jnp.zeros((16,), jnp.int32)
        vout[sl] = jnp.zeros((16,), jnp.float32)
        candval[sl] = jnp.full((16,), NEG, jnp.float32)

    def s2_body(t, c):
        m, qwin = _pick(cm, ca)
        _set1(qsel, t, qwin)
        _set1(candval, t, m)
        l = qwin & 15
        i = qwin >> 4
        _set1(qmax, qwin, NEG)
        _set1(qT, l * 128 + i, NEG)
        _recompute_lane(qT, cm, ca, l)
        return c
    lax.fori_loop(0, K, s2_body, jnp.int32(0))

    base = wid * Q
    for c0 in range(KPAD // 16):
        sl = pl.ds(c0 * 16, 16)
        qselg[sl] = qsel[sl] + base
        kvec[sl] = qsel[sl] * 128 + (lanes + c0 * 16)
    pltpu.async_copy(logits_hbm.at[qselg], rows, sem).wait()

    def cb_body(s, c):
        accv = rows[s, pl.ds(0, 16)]
        accj = jnp.zeros((16,), jnp.int32)
        for j in range(1, 32):
            v = rows[s, pl.ds(j * 16, 16)]
            upd = v > accv
            accj = jnp.where(upd, j, accj)
            accv = jnp.where(upd, v, accv)
        colv[pl.ds(s * 16, 16)] = accv
        colp[pl.ds(s * 16, 16)] = accj
        return c
    lax.fori_loop(0, KPAD, cb_body, jnp.int32(0))

    def s3_body(t, c):
        accv = candval[pl.ds(0, 16)]
        for c7 in range(1, KPAD // 16):
            accv = jnp.maximum(accv, candval[pl.ds(c7 * 16, 16)])
        m = jnp.max(accv)
        acck = jnp.full((16,), IBIG, jnp.int32)
        for c7 in range(KPAD // 16):
            sl = pl.ds(c7 * 16, 16)
            acck = jnp.minimum(
                acck, jnp.where(candval[sl] == m, kvec[sl], IBIG))
        key = jnp.min(acck)
        slot = key & 127
        _set1(vout, t, m)
        _set1(sout, t, slot)
        cvs = colv[pl.ds(slot * 16, 16)]
        cps = colp[pl.ds(slot * 16, 16)]
        cls = jnp.min(jnp.where(cvs == m, cps * 16 + lanes, IBIG))
        _set1(cout, t, cls)
        l2 = cls & 15
        off = (cls >> 4) << 4
        ch = rows[slot, pl.ds(off, 16)]
        rows[slot, pl.ds(off, 16)] = jnp.where(lanes == l2, NEG, ch)
        slotv = jnp.zeros((16,), jnp.int32) + slot
        g1 = plsc.load_gather(rows, [slotv, lanes * 16 + l2])
        g2 = plsc.load_gather(rows, [slotv, lanes * 16 + l2 + 256])
        upd = g2 > g1
        bv = jnp.where(upd, g2, g1)
        bj = jnp.where(upd, lanes + 16, lanes)
        nv = jnp.max(bv)
        nj = jnp.min(jnp.where(bv == nv, bj, IBIG))
        ncv = jnp.where(lanes == l2, nv, cvs)
        colv[pl.ds(slot * 16, 16)] = ncv
        colp[pl.ds(slot * 16, 16)] = jnp.where(lanes == l2, nj, cps)
        _set1(candval, slot, jnp.max(ncv))
        return c
    lax.fori_loop(0, K, s3_body, jnp.int32(0))

    tchunk = tsz[pl.ds((wid >> 4) << 4, 16)]
    scale = jnp.sum(jnp.where(lanes == (wid & 15), tchunk, 0.0))
    for c0 in range(KPAD // 16):
        sl = pl.ds(c0 * 16, 16)
        v = vout[sl]
        s_ = sout[sl]
        cls = cout[sl]
        q = plsc.load_gather(qsel, [s_])
        sco[sl] = 1.0 / (1.0 + jnp.exp(-v))
        labv[sl] = cls
        qidv[sl] = q
        cc = plsc.load_gather(boxes, [q])
        ww = plsc.load_gather(boxes, [q + Q])
        t1 = jnp.clip(cc - 0.5 * ww, 0.0, 1.0) * scale
        t2 = jnp.clip(cc + 0.5 * ww, 0.0, 1.0) * scale
        plsc.store_scatter(boxo, [(lanes + c0 * 16) * 2], t1)
        plsc.store_scatter(boxo, [(lanes + c0 * 16) * 2 + 1], t2)

    pltpu.sync_copy(sco, sco_hbm.at[pl.ds(wid * KPAD, KPAD)])
    pltpu.sync_copy(labv, lab_hbm.at[pl.ds(wid * KPAD, KPAD)])
    pltpu.sync_copy(qidv, qid_hbm.at[pl.ds(wid * KPAD, KPAD)])
    pltpu.sync_copy(boxo, box_hbm.at[pl.ds(wid * 2 * KPAD, 2 * KPAD)])


def _stage2(qmax, logits2d, boxes_flat, target_sizes):
    mesh = plsc.VectorSubcoreMesh(core_axis_name="c", subcore_axis_name="s")
    f32, i32 = jnp.float32, jnp.int32
    run = pl.kernel(
        _sc_body,
        out_type=[
            jax.ShapeDtypeStruct((B * KPAD,), f32),
            jax.ShapeDtypeStruct((B * KPAD,), i32),
            jax.ShapeDtypeStruct((B * KPAD,), i32),
            jax.ShapeDtypeStruct((B * 2 * KPAD,), f32),
        ],
        mesh=mesh,
        compiler_params=pltpu.CompilerParams(needs_layout_passes=False),
        scratch_types=[
            pltpu.VMEM((Q,), f32),
            pltpu.VMEM((Q,), f32),
            pltpu.VMEM((2 * Q,), f32),
            pltpu.VMEM((KPAD, C), f32),
            pltpu.VMEM((16,), f32),
            pltpu.VMEM((16,), i32),
            pltpu.VMEM((KPAD,), i32),
            pltpu.VMEM((KPAD,), i32),
            pltpu.VMEM((KPAD,), i32),
            pltpu.VMEM((KPAD,), f32),
            pltpu.VMEM((KPAD * 16,), f32),
            pltpu.VMEM((KPAD * 16,), i32),
            pltpu.VMEM((KPAD,), f32),
            pltpu.VMEM((KPAD,), i32),
            pltpu.VMEM((KPAD,), i32),
            pltpu.VMEM((32,), f32),
            pltpu.VMEM((KPAD,), f32),
            pltpu.VMEM((KPAD,), i32),
            pltpu.VMEM((KPAD,), i32),
            pltpu.VMEM((2 * KPAD,), f32),
            pltpu.SemaphoreType.DMA,
        ],
    )
    return run(qmax, logits2d, boxes_flat, target_sizes)


@jax.jit
def kernel(pred_logits, pred_boxes, target_sizes):
    qmax = _stage1(pred_logits)
    logits2d = pred_logits.reshape(B * Q, C)
    boxes_cw = jnp.transpose(pred_boxes, (0, 2, 1))
    sco, lab, qid, box = _stage2(
        qmax.reshape(-1), logits2d,
        boxes_cw.reshape(-1), target_sizes)
    scores = sco.reshape(B, KPAD)[:, :K]
    labels = lab.reshape(B, KPAD)[:, :K]
    tb_idx = qid.reshape(B, KPAD)[:, :K]
    boxes = box.reshape(B, KPAD, 2)[:, :K, :]
    return scores, labels, boxes, tb_idx


# speedup vs baseline: 1.7284x; 1.0027x over previous
import functools

import jax
import jax.numpy as jnp
from jax import lax
from jax.experimental import pallas as pl
from jax.experimental.pallas import tpu as pltpu
from jax.experimental.pallas import tpu_sc as plsc

B, Q, C = 32, 2048, 512
K = 100
KPAD = 112
NEG = float("-inf")
IBIG = 2**30


_BB = 4


def _tc_qmax_kernel(x_ref, m_ref):
    x = x_ref[...]
    m_ref[...] = jnp.max(x, axis=2)[:, None, :]


def _stage1(pred_logits):
    return pl.pallas_call(
        _tc_qmax_kernel,
        grid=(B // _BB,),
        in_specs=[pl.BlockSpec((_BB, Q, C), lambda b: (b, 0, 0))],
        out_specs=pl.BlockSpec((_BB, 1, Q), lambda b: (b, 0, 0)),
        out_shape=jax.ShapeDtypeStruct((B, 1, Q), jnp.float32),
    )(pred_logits)


_NCHUNK = Q // 16


def _iota16():
    return lax.broadcasted_iota(jnp.int32, (16,), 0)


def _vshuf(x, idx):
    dn = lax.GatherDimensionNumbers(
        offset_dims=(), collapsed_slice_dims=(0,), start_index_map=(0,))
    return lax.gather(x, idx[:, None], dn, (1,),
                      mode=lax.GatherScatterMode.PROMISE_IN_BOUNDS)


def _bcast_max(x):
    for s in (8, 4, 2, 1):
        x = jnp.maximum(x, _vshuf(x, _iota16() ^ s))
    return x


def _bcast_min(x):
    for s in (8, 4, 2, 1):
        x = jnp.minimum(x, _vshuf(x, _iota16() ^ s))
    return x


def _set1(ref, pos, val):
    off = (pos >> 4) << 4
    c = ref[pl.ds(off, 16)]
    ref[pl.ds(off, 16)] = jnp.where(_iota16() == (pos & 15), val, c)


def _scan128(ref, base):
    accv = ref[pl.ds(base, 16)]
    acci = _iota16()
    for jj in range(1, 8):
        v = ref[pl.ds(base + jj * 16, 16)]
        upd = v > accv
        acci = jnp.where(upd, _iota16() + jj * 16, acci)
        accv = jnp.where(upd, v, accv)
    m = _bcast_max(accv)
    pos = _bcast_min(jnp.where(accv == m, acci, IBIG))
    return m, pos


def _pick(cm_ref, ca_ref):
    cm = cm_ref[...]
    m = _bcast_max(cm)
    qv = ca_ref[...] * 16 + _iota16()
    qwin = jnp.min(jnp.where(cm == m, qv, IBIG))
    return m, qwin


def _recompute_lane(qT_ref, cm_ref, ca_ref, l):
    m, pos = _scan128(qT_ref, l * 128)
    lanes = _iota16()
    cm_ref[...] = jnp.where(lanes == l, m, cm_ref[...])
    ca_ref[...] = jnp.where(lanes == l, pos, ca_ref[...])


def _build_transpose(src_ref, dst_ref):
    def body(i, c):
        v = src_ref[pl.ds(i * 16, 16)]
        plsc.store_scatter(dst_ref, [_iota16() * 128 + i], v)
        return c
    lax.fori_loop(0, _NCHUNK, body, jnp.int32(0))


def _build_colmax(src_ref, cm_ref, ca_ref):
    def body(i, carry):
        accv, acci = carry
        v = src_ref[pl.ds(i * 16, 16)]
        upd = v > accv
        return (jnp.where(upd, v, accv), jnp.where(upd, i, acci))
    accv0 = src_ref[pl.ds(0, 16)]
    acci0 = jnp.zeros((16,), jnp.int32)
    accv, acci = lax.fori_loop(1, _NCHUNK, body, (accv0, acci0))
    cm_ref[...] = accv
    ca_ref[...] = acci


def _sc_body(qmax_hbm, logits_hbm, boxes_hbm, tsz_hbm,
             sco_hbm, lab_hbm, qid_hbm, box_hbm,
             qmax, qT, boxes, rows,
             cm, ca, qsel, qselg, kvec, candval, colv, colp,
             vout, sout, cout,
             tsz, sco, labv, qidv, boxo, sem):
    wid = lax.axis_index("s") * 2 + lax.axis_index("c")
    lanes = _iota16()

    pltpu.sync_copy(qmax_hbm.at[pl.ds(wid * Q, Q)], qmax)
    pltpu.sync_copy(boxes_hbm.at[pl.ds(wid * (2 * Q), 2 * Q)], boxes)
    pltpu.sync_copy(tsz_hbm, tsz)

    _build_transpose(qmax, qT)
    _build_colmax(qmax, cm, ca)

    for c0 in range(KPAD // 16):
        sl = pl.ds(c0 * 16, 16)
        qsel[sl] = jnp.zeros((16,), jnp.int32)
        sout[sl] = jnp.zeros((16,), jnp.int32)
        cout[sl] = jnp.zeros((16,), jnp.int32)
        vout[sl] = jnp.zeros((16,), jnp.float32)
        candval[sl] = jnp.full((16,), NEG, jnp.float32)

    def s2_body(t, c):
        m, qwin = _pick(cm, ca)
        _set1(qsel, t, qwin)
        _set1(candval, t, m)
        l = qwin & 15
        i = qwin >> 4
        _set1(qmax, qwin, NEG)
        _set1(qT, l * 128 + i, NEG)
        _recompute_lane(qT, cm, ca, l)
        return c
    lax.fori_loop(0, K, s2_body, jnp.int32(0))

    base = wid * Q
    for c0 in range(KPAD // 16):
        sl = pl.ds(c0 * 16, 16)
        qselg[sl] = qsel[sl] + base
        kvec[sl] = qsel[sl] * 128 + (lanes + c0 * 16)
    pltpu.async_copy(logits_hbm.at[qselg], rows, sem).wait()

    def cb_body(s, c):
        accv = rows[s, pl.ds(0, 16)]
        accj = jnp.zeros((16,), jnp.int32)
        for j in range(1, 32):
            v = rows[s, pl.ds(j * 16, 16)]
            upd = v > accv
            accj = jnp.where(upd, j, accj)
            accv = jnp.where(upd, v, accv)
        colv[pl.ds(s * 16, 16)] = accv
        colp[pl.ds(s * 16, 16)] = accj
        return c
    lax.fori_loop(0, KPAD, cb_body, jnp.int32(0))

    def s3_body(t, c):
        accv = candval[pl.ds(0, 16)]
        for c7 in range(1, KPAD // 16):
            accv = jnp.maximum(accv, candval[pl.ds(c7 * 16, 16)])
        m = _bcast_max(accv)
        acck = jnp.full((16,), IBIG, jnp.int32)
        for c7 in range(KPAD // 16):
            sl = pl.ds(c7 * 16, 16)
            acck = jnp.minimum(
                acck, jnp.where(candval[sl] == m, kvec[sl], IBIG))
        key = jnp.min(acck)
        slot = key & 127
        _set1(vout, t, m)
        _set1(sout, t, slot)
        cvs = colv[pl.ds(slot * 16, 16)]
        cps = colp[pl.ds(slot * 16, 16)]
        cls = jnp.min(jnp.where(cvs == m, cps * 16 + lanes, IBIG))
        _set1(cout, t, cls)
        l2 = cls & 15
        off = (cls >> 4) << 4
        ch = rows[slot, pl.ds(off, 16)]
        rows[slot, pl.ds(off, 16)] = jnp.where(lanes == l2, NEG, ch)
        slotv = jnp.zeros((16,), jnp.int32) + slot
        g1 = plsc.load_gather(rows, [slotv, lanes * 16 + l2])
        g2 = plsc.load_gather(rows, [slotv, lanes * 16 + l2 + 256])
        upd = g2 > g1
        bv = jnp.where(upd, g2, g1)
        bj = jnp.where(upd, lanes + 16, lanes)
        nv = _bcast_max(bv)
        nj = _bcast_min(jnp.where(bv == nv, bj, IBIG))
        ncv = jnp.where(lanes == l2, nv, cvs)
        colv[pl.ds(slot * 16, 16)] = ncv
        colp[pl.ds(slot * 16, 16)] = jnp.where(lanes == l2, nj, cps)
        _set1(candval, slot, _bcast_max(ncv))
        return c
    lax.fori_loop(0, K, s3_body, jnp.int32(0))

    tchunk = tsz[pl.ds((wid >> 4) << 4, 16)]
    scale = jnp.sum(jnp.where(lanes == (wid & 15), tchunk, 0.0))
    for c0 in range(KPAD // 16):
        sl = pl.ds(c0 * 16, 16)
        v = vout[sl]
        s_ = sout[sl]
        cls = cout[sl]
        q = plsc.load_gather(qsel, [s_])
        sco[sl] = 1.0 / (1.0 + jnp.exp(-v))
        labv[sl] = cls
        qidv[sl] = q
        cc = plsc.load_gather(boxes, [q])
        ww = plsc.load_gather(boxes, [q + Q])
        t1 = jnp.clip(cc - 0.5 * ww, 0.0, 1.0) * scale
        t2 = jnp.clip(cc + 0.5 * ww, 0.0, 1.0) * scale
        plsc.store_scatter(boxo, [(lanes + c0 * 16) * 2], t1)
        plsc.store_scatter(boxo, [(lanes + c0 * 16) * 2 + 1], t2)

    pltpu.sync_copy(sco, sco_hbm.at[pl.ds(wid * KPAD, KPAD)])
    pltpu.sync_copy(labv, lab_hbm.at[pl.ds(wid * KPAD, KPAD)])
    pltpu.sync_copy(qidv, qid_hbm.at[pl.ds(wid * KPAD, KPAD)])
    pltpu.sync_copy(boxo, box_hbm.at[pl.ds(wid * 2 * KPAD, 2 * KPAD)])


def _stage2(qmax, logits2d, boxes_flat, target_sizes):
    mesh = plsc.VectorSubcoreMesh(core_axis_name="c", subcore_axis_name="s")
    f32, i32 = jnp.float32, jnp.int32
    run = pl.kernel(
        _sc_body,
        out_type=[
            jax.ShapeDtypeStruct((B * KPAD,), f32),
            jax.ShapeDtypeStruct((B * KPAD,), i32),
            jax.ShapeDtypeStruct((B * KPAD,), i32),
            jax.ShapeDtypeStruct((B * 2 * KPAD,), f32),
        ],
        mesh=mesh,
        compiler_params=pltpu.CompilerParams(needs_layout_passes=False),
        scratch_types=[
            pltpu.VMEM((Q,), f32),
            pltpu.VMEM((Q,), f32),
            pltpu.VMEM((2 * Q,), f32),
            pltpu.VMEM((KPAD, C), f32),
            pltpu.VMEM((16,), f32),
            pltpu.VMEM((16,), i32),
            pltpu.VMEM((KPAD,), i32),
            pltpu.VMEM((KPAD,), i32),
            pltpu.VMEM((KPAD,), i32),
            pltpu.VMEM((KPAD,), f32),
            pltpu.VMEM((KPAD * 16,), f32),
            pltpu.VMEM((KPAD * 16,), i32),
            pltpu.VMEM((KPAD,), f32),
            pltpu.VMEM((KPAD,), i32),
            pltpu.VMEM((KPAD,), i32),
            pltpu.VMEM((32,), f32),
            pltpu.VMEM((KPAD,), f32),
            pltpu.VMEM((KPAD,), i32),
            pltpu.VMEM((KPAD,), i32),
            pltpu.VMEM((2 * KPAD,), f32),
            pltpu.SemaphoreType.DMA,
        ],
    )
    return run(qmax, logits2d, boxes_flat, target_sizes)


@jax.jit
def kernel(pred_logits, pred_boxes, target_sizes):
    qmax = _stage1(pred_logits)
    logits2d = pred_logits.reshape(B * Q, C)
    boxes_cw = jnp.transpose(pred_boxes, (0, 2, 1))
    sco, lab, qid, box = _stage2(
        qmax.reshape(-1), logits2d,
        boxes_cw.reshape(-1), target_sizes)
    scores = sco.reshape(B, KPAD)[:, :K]
    labels = lab.reshape(B, KPAD)[:, :K]
    tb_idx = qid.reshape(B, KPAD)[:, :K]
    boxes = box.reshape(B, KPAD, 2)[:, :K, :]
    return scores, labels, boxes, tb_idx


# speedup vs baseline: 1.8514x; 1.0712x over previous
import functools

import jax
import jax.numpy as jnp
from jax import lax
from jax.experimental import pallas as pl
from jax.experimental.pallas import tpu as pltpu
from jax.experimental.pallas import tpu_sc as plsc

B, Q, C = 32, 2048, 512
K = 100
KPAD = 112
NEG = float("-inf")
IBIG = 2**30


_BB = 4


def _tc_qmax_kernel(x_ref, m_ref):
    x = x_ref[...]
    m_ref[...] = jnp.max(x, axis=2)[:, None, :]


def _stage1(pred_logits):
    return pl.pallas_call(
        _tc_qmax_kernel,
        grid=(B // _BB,),
        in_specs=[pl.BlockSpec((_BB, Q, C), lambda b: (b, 0, 0))],
        out_specs=pl.BlockSpec((_BB, 1, Q), lambda b: (b, 0, 0)),
        out_shape=jax.ShapeDtypeStruct((B, 1, Q), jnp.float32),
    )(pred_logits)


_NCHUNK = Q // 16


def _iota16():
    return lax.broadcasted_iota(jnp.int32, (16,), 0)


def _vshuf(x, idx):
    dn = lax.GatherDimensionNumbers(
        offset_dims=(), collapsed_slice_dims=(0,), start_index_map=(0,))
    return lax.gather(x, idx[:, None], dn, (1,),
                      mode=lax.GatherScatterMode.PROMISE_IN_BOUNDS)


def _bcast_max(x):
    for s in (8, 4, 2, 1):
        x = jnp.maximum(x, _vshuf(x, _iota16() ^ s))
    return x


def _bcast_min(x):
    for s in (8, 4, 2, 1):
        x = jnp.minimum(x, _vshuf(x, _iota16() ^ s))
    return x


def _set1(ref, pos, val):
    off = (pos >> 4) << 4
    c = ref[pl.ds(off, 16)]
    ref[pl.ds(off, 16)] = jnp.where(_iota16() == (pos & 15), val, c)


def _scan128(ref, base):
    accv = ref[pl.ds(base, 16)]
    acci = _iota16()
    for jj in range(1, 8):
        v = ref[pl.ds(base + jj * 16, 16)]
        upd = v > accv
        acci = jnp.where(upd, _iota16() + jj * 16, acci)
        accv = jnp.where(upd, v, accv)
    m = _bcast_max(accv)
    pos = _bcast_min(jnp.where(accv == m, acci, IBIG))
    return m, pos


def _pick(cm_ref, ca_ref):
    cm = cm_ref[...]
    m = _bcast_max(cm)
    qv = ca_ref[...] * 16 + _iota16()
    qwin = jnp.min(jnp.where(cm == m, qv, IBIG))
    return m, qwin


def _recompute_lane(qT_ref, cm_ref, ca_ref, l):
    m, pos = _scan128(qT_ref, l * 128)
    lanes = _iota16()
    cm_ref[...] = jnp.where(lanes == l, m, cm_ref[...])
    ca_ref[...] = jnp.where(lanes == l, pos, ca_ref[...])


def _build_transpose(src_ref, dst_ref):
    def body(i, c):
        v = src_ref[pl.ds(i * 16, 16)]
        plsc.store_scatter(dst_ref, [_iota16() * 128 + i], v)
        return c
    lax.fori_loop(0, _NCHUNK, body, jnp.int32(0))


def _build_colmax(src_ref, cm_ref, ca_ref):
    def body(i, carry):
        accv, acci = carry
        v = src_ref[pl.ds(i * 16, 16)]
        upd = v > accv
        return (jnp.where(upd, v, accv), jnp.where(upd, i, acci))
    accv0 = src_ref[pl.ds(0, 16)]
    acci0 = jnp.zeros((16,), jnp.int32)
    accv, acci = lax.fori_loop(1, _NCHUNK, body, (accv0, acci0))
    cm_ref[...] = accv
    ca_ref[...] = acci


def _sc_body(qmax_hbm, logits_hbm, boxes_hbm, tsz_hbm,
             sco_hbm, lab_hbm, qid_hbm, box_hbm,
             qmax, qT, boxes, rows,
             cm, ca, qsel, qselg, kvec, candval, colv, colp,
             vout, sout, cout,
             tsz, sco, labv, qidv, boxo, sem):
    wid = lax.axis_index("s") * 2 + lax.axis_index("c")
    lanes = _iota16()

    pltpu.sync_copy(qmax_hbm.at[pl.ds(wid * Q, Q)], qmax)
    pltpu.sync_copy(boxes_hbm.at[pl.ds(wid * (2 * Q), 2 * Q)], boxes)
    pltpu.sync_copy(tsz_hbm, tsz)

    _build_transpose(qmax, qT)
    _build_colmax(qmax, cm, ca)

    for c0 in range(KPAD // 16):
        sl = pl.ds(c0 * 16, 16)
        qsel[sl] = jnp.zeros((16,), jnp.int32)
        sout[sl] = jnp.zeros((16,), jnp.int32)
        cout[sl] = jnp.zeros((16,), jnp.int32)
        vout[sl] = jnp.zeros((16,), jnp.float32)
        candval[sl] = jnp.full((16,), NEG, jnp.float32)

    def s2_body(t, c):
        m, qwin = _pick(cm, ca)
        _set1(qsel, t, qwin)
        _set1(candval, t, m)
        l = qwin & 15
        i = qwin >> 4
        _set1(qmax, qwin, NEG)
        _set1(qT, l * 128 + i, NEG)
        _recompute_lane(qT, cm, ca, l)
        return c
    lax.fori_loop(0, K, s2_body, jnp.int32(0))

    base = wid * Q
    for c0 in range(KPAD // 16):
        sl = pl.ds(c0 * 16, 16)
        qselg[sl] = qsel[sl] + base
        kvec[sl] = qsel[sl] * 128 + (lanes + c0 * 16)
    pltpu.async_copy(logits_hbm.at[qselg], rows, sem).wait()

    def cb_body(s, c):
        accv = rows[s, pl.ds(0, 16)]
        accj = jnp.zeros((16,), jnp.int32)
        for j in range(1, 32):
            v = rows[s, pl.ds(j * 16, 16)]
            upd = v > accv
            accj = jnp.where(upd, j, accj)
            accv = jnp.where(upd, v, accv)
        colv[pl.ds(s * 16, 16)] = accv
        colp[pl.ds(s * 16, 16)] = accj
        return c
    lax.fori_loop(0, KPAD, cb_body, jnp.int32(0))

    def s3_body(t, c):
        accv = candval[pl.ds(0, 16)]
        for c7 in range(1, KPAD // 16):
            accv = jnp.maximum(accv, candval[pl.ds(c7 * 16, 16)])
        m = _bcast_max(accv)
        acck = jnp.full((16,), IBIG, jnp.int32)
        for c7 in range(KPAD // 16):
            sl = pl.ds(c7 * 16, 16)
            acck = jnp.minimum(
                acck, jnp.where(candval[sl] == m, kvec[sl], IBIG))
        key = jnp.min(acck)
        slot = key & 127
        _set1(vout, t, m)
        _set1(sout, t, slot)
        cvs = colv[pl.ds(slot * 16, 16)]
        cps = colp[pl.ds(slot * 16, 16)]
        cls = jnp.min(jnp.where(cvs == m, cps * 16 + lanes, IBIG))
        _set1(cout, t, cls)
        l2 = cls & 15
        off = (cls >> 4) << 4
        ch = rows[slot, pl.ds(off, 16)]
        rows[slot, pl.ds(off, 16)] = jnp.where(lanes == l2, NEG, ch)
        slotv = jnp.zeros((16,), jnp.int32) + slot
        g1 = plsc.load_gather(rows, [slotv, lanes * 16 + l2])
        g2 = plsc.load_gather(rows, [slotv, lanes * 16 + l2 + 256])
        upd = g2 > g1
        bv = jnp.where(upd, g2, g1)
        bj = jnp.where(upd, lanes + 16, lanes)
        nv = _bcast_max(bv)
        nj = _bcast_min(jnp.where(bv == nv, bj, IBIG))
        ncv = jnp.where(lanes == l2, nv, cvs)
        colv[pl.ds(slot * 16, 16)] = ncv
        colp[pl.ds(slot * 16, 16)] = jnp.where(lanes == l2, nj, cps)
        _set1(candval, slot, _bcast_max(ncv))
        return c
    lax.fori_loop(0, K, s3_body, jnp.int32(0))

    tchunk = tsz[pl.ds((wid >> 4) << 4, 16)]
    scale = jnp.sum(jnp.where(lanes == (wid & 15), tchunk, 0.0))
    for c0 in range(KPAD // 16):
        sl = pl.ds(c0 * 16, 16)
        v = vout[sl]
        s_ = sout[sl]
        cls = cout[sl]
        q = plsc.load_gather(qsel, [s_])
        sco[sl] = 1.0 / (1.0 + jnp.exp(-v))
        labv[sl] = cls
        qidv[sl] = q
        cc = plsc.load_gather(boxes, [q])
        ww = plsc.load_gather(boxes, [q + Q])
        t1 = jnp.clip(cc - 0.5 * ww, 0.0, 1.0) * scale
        t2 = jnp.clip(cc + 0.5 * ww, 0.0, 1.0) * scale
        plsc.store_scatter(boxo, [(lanes + c0 * 16) * 2], t1)
        plsc.store_scatter(boxo, [(lanes + c0 * 16) * 2 + 1], t2)

    pltpu.sync_copy(sco, sco_hbm.at[wid])
    pltpu.sync_copy(labv, lab_hbm.at[wid])
    pltpu.sync_copy(qidv, qid_hbm.at[wid])
    pltpu.sync_copy(boxo, box_hbm.at[wid])


def _stage2(qmax, logits2d, boxes_flat, target_sizes):
    mesh = plsc.VectorSubcoreMesh(core_axis_name="c", subcore_axis_name="s")
    f32, i32 = jnp.float32, jnp.int32
    run = pl.kernel(
        _sc_body,
        out_type=[
            jax.ShapeDtypeStruct((B, KPAD), f32),
            jax.ShapeDtypeStruct((B, KPAD), i32),
            jax.ShapeDtypeStruct((B, KPAD), i32),
            jax.ShapeDtypeStruct((B, 2 * KPAD), f32),
        ],
        mesh=mesh,
        compiler_params=pltpu.CompilerParams(needs_layout_passes=False),
        scratch_types=[
            pltpu.VMEM((Q,), f32),
            pltpu.VMEM((Q,), f32),
            pltpu.VMEM((2 * Q,), f32),
            pltpu.VMEM((KPAD, C), f32),
            pltpu.VMEM((16,), f32),
            pltpu.VMEM((16,), i32),
            pltpu.VMEM((KPAD,), i32),
            pltpu.VMEM((KPAD,), i32),
            pltpu.VMEM((KPAD,), i32),
            pltpu.VMEM((KPAD,), f32),
            pltpu.VMEM((KPAD * 16,), f32),
            pltpu.VMEM((KPAD * 16,), i32),
            pltpu.VMEM((KPAD,), f32),
            pltpu.VMEM((KPAD,), i32),
            pltpu.VMEM((KPAD,), i32),
            pltpu.VMEM((32,), f32),
            pltpu.VMEM((KPAD,), f32),
            pltpu.VMEM((KPAD,), i32),
            pltpu.VMEM((KPAD,), i32),
            pltpu.VMEM((2 * KPAD,), f32),
            pltpu.SemaphoreType.DMA,
        ],
    )
    return run(qmax, logits2d, boxes_flat, target_sizes)


@jax.jit
def kernel(pred_logits, pred_boxes, target_sizes):
    qmax = _stage1(pred_logits)
    logits2d = pred_logits.reshape(B * Q, C)
    boxes_cw = jnp.transpose(pred_boxes, (0, 2, 1))
    sco, lab, qid, box = _stage2(
        qmax.reshape(-1), logits2d,
        boxes_cw.reshape(-1), target_sizes)
    scores = sco[:, :K]
    labels = lab[:, :K]
    tb_idx = qid[:, :K]
    boxes = box.reshape(B, KPAD, 2)[:, :K, :]
    return scores, labels, boxes, tb_idx
